# baseline (speedup 1.0000x reference)
import functools

import jax
import jax.numpy as jnp
from jax import lax
from jax.experimental import pallas as pl
from jax.experimental.pallas import tpu as pltpu
from jax.experimental.pallas import tpu_sc as plsc

NUM_SUBCORES = 16
LANES = 16

N_STRUCTURES = 1000
N_STRUCT_PAD = 1024
ACC_ROWS = 8
ROW_STRIDE = N_STRUCT_PAD + 1
SPECIES_PAD = 128
UNROLL = 4
SUBCHUNKS = 4


def _tree_add(vs):
    while len(vs) > 1:
        vs = [a + b for a, b in zip(vs[::2], vs[1::2])] + (
            [vs[-1]] if len(vs) % 2 else [])
    return vs[0]


def _sc_total(e, zb, scale, offset, *, chunk):
    sub = chunk // SUBCHUNKS
    nblk = N_STRUCT_PAD // LANES
    cols = N_STRUCT_PAD // NUM_SUBCORES
    mesh = plsc.VectorSubcoreMesh(core_axis_name="c", subcore_axis_name="s",
                                  num_cores=1)

    @functools.partial(
        pl.kernel,
        out_type=jax.ShapeDtypeStruct((1, N_STRUCT_PAD), jnp.float32),
        mesh=mesh,
        scratch_types=[
            pltpu.VMEM((chunk,), jnp.float32),
            pltpu.VMEM((chunk,), jnp.int32),
            pltpu.VMEM((SPECIES_PAD,), jnp.float32),
            pltpu.VMEM((SPECIES_PAD,), jnp.float32),
            pltpu.VMEM((ACC_ROWS * ROW_STRIDE,), jnp.float32),
            pltpu.VMEM((N_STRUCT_PAD,), jnp.float32),
            pltpu.VMEM((NUM_SUBCORES, cols), jnp.float32),
            pltpu.VMEM((cols,), jnp.float32),
            pltpu.VMEM_SHARED((NUM_SUBCORES, N_STRUCT_PAD), jnp.float32),
            [pltpu.SemaphoreType.DMA] * SUBCHUNKS,
        ],
        compiler_params=pltpu.CompilerParams(needs_layout_passes=False),
    )
    def body(e_hbm, zb_hbm, sc_hbm, of_hbm, out_hbm,
             e_v, zb_v, sc_v, of_v, acc, acc1d, buf, outv, shared, sems):
        sid = lax.axis_index("s")
        base = sid * chunk
        copies = []
        for k in range(SUBCHUNKS):
            o = k * sub
            copies.append([
                pltpu.async_copy(e_hbm.at[pl.ds(base + o, sub)],
                                 e_v.at[pl.ds(o, sub)], sems[k]),
                pltpu.async_copy(zb_hbm.at[pl.ds(base + o, sub)],
                                 zb_v.at[pl.ds(o, sub)], sems[k]),
            ])
        tab_copies = [
            pltpu.async_copy(sc_hbm, sc_v, sems[0]),
            pltpu.async_copy(of_hbm, of_v, sems[0]),
        ]

        zeros = jnp.zeros((LANES,), jnp.float32)

        def zero_body(j, carry):
            o = j * LANES
            for r in range(ACC_ROWS):
                acc[pl.ds(r * ROW_STRIDE + o, LANES)] = zeros
            return carry

        lax.fori_loop(0, nblk, zero_body, 0)

        lanes = lax.iota(jnp.int32, LANES)

        for k in range(SUBCHUNKS):
            for c in copies[k]:
                c.wait()
            if k == 0:
                for c in tab_copies:
                    c.wait()

            @plsc.parallel_loop(k * sub, (k + 1) * sub, step=LANES,
                                unroll=UNROLL)
            def _main(i):
                e16 = e_v[pl.ds(i, LANES)]
                zb16 = zb_v[pl.ds(i, LANES)]
                z16 = lax.bitwise_and(zb16, SPECIES_PAD - 1)
                b16 = lax.shift_right_logical(zb16, 7)
                rot = lax.shift_right_logical(i, 4)
                rows = lax.bitwise_and(lanes + rot, ACC_ROWS - 1)
                sv = plsc.load_gather(sc_v, [z16])
                ov = plsc.load_gather(of_v, [z16])
                plsc.addupdate_scatter(acc, [rows * ROW_STRIDE + b16],
                                       e16 * sv + ov)

        @plsc.parallel_loop(0, N_STRUCT_PAD, step=LANES, unroll=2)
        def _reduce(o):
            vs = [acc[pl.ds(r * ROW_STRIDE + o, LANES)]
                  for r in range(ACC_ROWS)]
            acc1d[pl.ds(o, LANES)] = _tree_add(vs)

        pltpu.sync_copy(acc1d, shared.at[sid])
        plsc.subcore_barrier()

        cbase = sid * cols
        row_copies = [
            pltpu.async_copy(shared.at[r, pl.ds(cbase, cols)], buf.at[r],
                             sems[0])
            for r in range(NUM_SUBCORES)
        ]
        for c in row_copies:
            c.wait()
        for cb in range(cols // LANES):
            o = cb * LANES
            vs = [buf[r, pl.ds(o, LANES)] for r in range(NUM_SUBCORES)]
            outv[pl.ds(o, LANES)] = _tree_add(vs)
        pltpu.sync_copy(outv, out_hbm.at[0, pl.ds(cbase, cols)])

    return body(e, zb, scale, offset)


def kernel(local_energies, Z, batch, scale, offset):
    n = local_energies.shape[0]
    per = LANES * UNROLL * SUBCHUNKS
    chunk = -(-n // (NUM_SUBCORES * per)) * per
    padn = NUM_SUBCORES * chunk
    e_p = jnp.pad(local_energies, (0, padn - n))
    zb = Z + batch * SPECIES_PAD
    zb_p = jnp.pad(zb, (0, padn - n),
                   constant_values=N_STRUCTURES * SPECIES_PAD)
    sc_p = jnp.pad(scale, (0, SPECIES_PAD - scale.shape[0]))
    of_p = jnp.pad(offset, (0, SPECIES_PAD - offset.shape[0]))
    total = _sc_total(e_p, zb_p, sc_p, of_p, chunk=chunk)
    return total[0, :N_STRUCTURES]
